# baseline (speedup 1.0000x reference)
import functools
import math

import jax
import jax.numpy as jnp
import numpy as np
from jax import lax
from jax.experimental import pallas as pl
from jax.experimental.pallas import tpu as pltpu
from jax.experimental.pallas import tpu_sc as plsc

_EMBED = 128
_MAX_LEN = 30
_LANES = 16
_NC, _NS = 2, 16
_NW = _NC * _NS


def _pe_table() -> np.ndarray:
    position = np.arange(_MAX_LEN, dtype=np.float32)[:, None]
    div_term = np.exp(
        np.arange(0, _EMBED, 2, dtype=np.float32) * -(math.log(10000.0) / _EMBED)
    )
    pe = np.zeros((_MAX_LEN, _EMBED), dtype=np.float32)
    pe[:, 0::2] = np.sin(position * div_term)
    pe[:, 1::2] = np.cos(position * div_term)
    return pe


_PE = _pe_table()

_PER_W = 104
_CHUNKS = ((0, 16), (16, 40), (56, 40), (96, 8))


@functools.lru_cache(maxsize=None)
def _build_sc_kernel(n: int, half_l: int):
    mesh = plsc.VectorSubcoreMesh(core_axis_name="c", subcore_axis_name="s")

    @functools.partial(
        pl.kernel,
        out_type=jax.ShapeDtypeStruct((n, _EMBED), jnp.float32),
        mesh=mesh,
        scratch_types=[
            pltpu.VMEM((_PER_W,), jnp.int32),
            pltpu.VMEM((_EMBED,), jnp.float32),
            pltpu.VMEM((_EMBED,), jnp.float32),
            pltpu.VMEM((_PER_W, _EMBED), jnp.float32),
        ]
        + [pltpu.SemaphoreType.DMA] * (2 * len(_CHUNKS) + 2),
    )
    def sc_kernel(idx_hbm, pe_hbm, seg_hbm, table_hbm, out_hbm,
                  idx_v, pe_v, seg_v, rows_v, *sems):
        nch = len(_CHUNKS)
        gsems, wsems, bsems = sems[:nch], sems[nch:2 * nch], sems[2 * nch:]
        wid = lax.axis_index("s") * _NC + lax.axis_index("c")
        b = wid // 2
        base = b * (2 * half_l) + (wid % 2) * (2 * half_l - _PER_W)
        pe_cp = pltpu.async_copy(pe_hbm.at[b], pe_v, bsems[0])
        seg_cp = pltpu.async_copy(seg_hbm.at[0], seg_v, bsems[1])
        pltpu.sync_copy(idx_hbm.at[pl.ds(base, _PER_W)], idx_v)
        gathers = [
            pltpu.async_copy(
                table_hbm.at[idx_v.at[pl.ds(off, cnt)]],
                rows_v.at[pl.ds(off, cnt)],
                gsems[k],
            )
            for k, (off, cnt) in enumerate(_CHUNKS)
        ]
        pe_cp.wait()
        seg_cp.wait()
        bias = [
            pe_v[pl.ds(j * _LANES, _LANES)] + seg_v[pl.ds(j * _LANES, _LANES)]
            for j in range(_EMBED // _LANES)
        ]

        def add_row(i, carry):
            for j in range(_EMBED // _LANES):
                sl = pl.ds(j * _LANES, _LANES)
                rows_v[i, sl] = rows_v[i, sl] + bias[j]
            return carry

        writes = []
        for k, (off, cnt) in enumerate(_CHUNKS):
            gathers[k].wait()
            lax.fori_loop(off, off + cnt, add_row, 0, unroll=8)
            writes.append(
                pltpu.async_copy(
                    rows_v.at[pl.ds(off, cnt)],
                    out_hbm.at[pl.ds(base + off, cnt)],
                    wsems[k],
                )
            )
        for w in writes:
            w.wait()

    return sc_kernel


def kernel(sequence, token_table, segment_weight):
    B, L = sequence.shape
    idx = sequence.astype(jnp.int32).reshape(-1)
    pe = jnp.asarray(_PE[:B])
    out = _build_sc_kernel(B * L, L // 2)(idx, pe, segment_weight, token_table)
    return out.reshape(B, L, _EMBED)


# speedup vs baseline: 1.0224x; 1.0224x over previous
import functools
import math

import jax
import jax.numpy as jnp
import numpy as np
from jax import lax
from jax.experimental import pallas as pl
from jax.experimental.pallas import tpu as pltpu
from jax.experimental.pallas import tpu_sc as plsc

_EMBED = 128
_MAX_LEN = 30
_LANES = 16
_NC, _NS = 2, 16
_NW = _NC * _NS


def _pe_table() -> np.ndarray:
    position = np.arange(_MAX_LEN, dtype=np.float32)[:, None]
    div_term = np.exp(
        np.arange(0, _EMBED, 2, dtype=np.float32) * -(math.log(10000.0) / _EMBED)
    )
    pe = np.zeros((_MAX_LEN, _EMBED), dtype=np.float32)
    pe[:, 0::2] = np.sin(position * div_term)
    pe[:, 1::2] = np.cos(position * div_term)
    return pe


_PE = _pe_table()

_PER_W = 104
_CHUNKS = ((0, 16), (16, 40), (56, 40), (96, 8))


@functools.lru_cache(maxsize=None)
def _build_sc_kernel(n: int, half_l: int):
    mesh = plsc.VectorSubcoreMesh(core_axis_name="c", subcore_axis_name="s")

    @functools.partial(
        pl.kernel,
        out_type=jax.ShapeDtypeStruct((n, _EMBED), jnp.float32),
        mesh=mesh,
        scratch_types=[
            pltpu.VMEM((_PER_W,), jnp.int32),
            pltpu.VMEM((_EMBED,), jnp.float32),
            pltpu.VMEM((_PER_W, _EMBED), jnp.float32),
        ]
        + [pltpu.SemaphoreType.DMA] * (2 * len(_CHUNKS) + 2),
    )
    def sc_kernel(idx_hbm, bias_hbm, table_hbm, out_hbm,
                  idx_v, bias_v, rows_v, *sems):
        nch = len(_CHUNKS)
        gsems, wsems, bsems = sems[:nch], sems[nch:2 * nch], sems[2 * nch:]
        wid = lax.axis_index("s") * _NC + lax.axis_index("c")
        b = wid // 2
        base = b * (2 * half_l) + (wid % 2) * (2 * half_l - _PER_W)
        bias_cp = pltpu.async_copy(bias_hbm.at[b], bias_v, bsems[0])
        idx_cp = pltpu.async_copy(idx_hbm.at[pl.ds(base, _PER_W)], idx_v, bsems[1])
        idx_cp.wait()
        gathers = [
            pltpu.async_copy(
                table_hbm.at[idx_v.at[pl.ds(off, cnt)]],
                rows_v.at[pl.ds(off, cnt)],
                gsems[k],
            )
            for k, (off, cnt) in enumerate(_CHUNKS)
        ]
        bias_cp.wait()
        bias = [bias_v[pl.ds(j * _LANES, _LANES)] for j in range(_EMBED // _LANES)]

        def add_row(i, carry):
            for j in range(_EMBED // _LANES):
                sl = pl.ds(j * _LANES, _LANES)
                rows_v[i, sl] = rows_v[i, sl] + bias[j]
            return carry

        writes = []
        for k, (off, cnt) in enumerate(_CHUNKS):
            gathers[k].wait()
            lax.fori_loop(off, off + cnt, add_row, 0, unroll=4)
            writes.append(
                pltpu.async_copy(
                    rows_v.at[pl.ds(off, cnt)],
                    out_hbm.at[pl.ds(base + off, cnt)],
                    wsems[k],
                )
            )
        for w in writes:
            w.wait()

    return sc_kernel


def kernel(sequence, token_table, segment_weight):
    B, L = sequence.shape
    idx = sequence.astype(jnp.int32).reshape(-1)
    bias2d = jnp.asarray(_PE[:B]) + segment_weight
    out = _build_sc_kernel(B * L, L // 2)(idx, bias2d, token_table)
    return out.reshape(B, L, _EMBED)


# speedup vs baseline: 1.0323x; 1.0097x over previous
import functools
import math

import jax
import jax.numpy as jnp
import numpy as np
from jax import lax
from jax.experimental import pallas as pl
from jax.experimental.pallas import tpu as pltpu
from jax.experimental.pallas import tpu_sc as plsc

_EMBED = 128
_MAX_LEN = 30
_LANES = 16
_NC, _NS = 2, 16
_NW = _NC * _NS


def _pe_table() -> np.ndarray:
    position = np.arange(_MAX_LEN, dtype=np.float32)[:, None]
    div_term = np.exp(
        np.arange(0, _EMBED, 2, dtype=np.float32) * -(math.log(10000.0) / _EMBED)
    )
    pe = np.zeros((_MAX_LEN, _EMBED), dtype=np.float32)
    pe[:, 0::2] = np.sin(position * div_term)
    pe[:, 1::2] = np.cos(position * div_term)
    return pe


_PE = _pe_table()

_PER_W = 104
_CHUNKS = ((0, 16), (16, 40), (56, 40), (96, 8))


@functools.lru_cache(maxsize=None)
def _build_sc_kernel(n: int, half_l: int):
    mesh = plsc.VectorSubcoreMesh(core_axis_name="c", subcore_axis_name="s")

    @functools.partial(
        pl.kernel,
        out_type=jax.ShapeDtypeStruct((n, _EMBED), jnp.float32),
        mesh=mesh,
        scratch_types=[
            pltpu.VMEM((_PER_W,), jnp.int32),
            pltpu.VMEM((_EMBED,), jnp.float32),
            pltpu.VMEM((_PER_W, _EMBED), jnp.float32),
        ]
        + [pltpu.SemaphoreType.DMA] * (2 * len(_CHUNKS) + 2),
    )
    def sc_kernel(idx_hbm, bias_hbm, table_hbm, out_hbm,
                  idx_v, bias_v, rows_v, *sems):
        nch = len(_CHUNKS)
        gsems, wsems, bsems = sems[:nch], sems[nch:2 * nch], sems[2 * nch:]
        wid = lax.axis_index("s") * _NC + lax.axis_index("c")
        b = wid // 2
        base = b * (2 * half_l) + (wid % 2) * (2 * half_l - _PER_W)
        bias_cp = pltpu.async_copy(bias_hbm.at[b], bias_v, bsems[0])
        idx_cp = pltpu.async_copy(idx_hbm.at[pl.ds(base, _PER_W)], idx_v, bsems[1])
        idx_cp.wait()
        gathers = [
            pltpu.async_copy(
                table_hbm.at[idx_v.at[pl.ds(off, cnt)]],
                rows_v.at[pl.ds(off, cnt)],
                gsems[k],
            )
            for k, (off, cnt) in enumerate(_CHUNKS)
        ]
        bias_cp.wait()
        bias = [bias_v[pl.ds(j * _LANES, _LANES)] for j in range(_EMBED // _LANES)]

        def add_row(i, carry):
            for j in range(_EMBED // _LANES):
                sl = pl.ds(j * _LANES, _LANES)
                rows_v[i, sl] = rows_v[i, sl] + bias[j]
            return carry

        writes = []
        for k, (off, cnt) in enumerate(_CHUNKS):
            gathers[k].wait()
            lax.fori_loop(off, off + cnt, add_row, 0, unroll=2)
            writes.append(
                pltpu.async_copy(
                    rows_v.at[pl.ds(off, cnt)],
                    out_hbm.at[pl.ds(base + off, cnt)],
                    wsems[k],
                )
            )
        for w in writes:
            w.wait()

    return sc_kernel


def kernel(sequence, token_table, segment_weight):
    B, L = sequence.shape
    idx = sequence.astype(jnp.int32).reshape(-1)
    bias2d = jnp.asarray(_PE[:B]) + segment_weight
    out = _build_sc_kernel(B * L, L // 2)(idx, bias2d, token_table)
    return out.reshape(B, L, _EMBED)


# speedup vs baseline: 1.0481x; 1.0153x over previous
import functools
import math

import jax
import jax.numpy as jnp
import numpy as np
from jax import lax
from jax.experimental import pallas as pl
from jax.experimental.pallas import tpu as pltpu
from jax.experimental.pallas import tpu_sc as plsc

_EMBED = 128
_MAX_LEN = 30
_LANES = 16
_NC, _NS = 2, 16
_NW = _NC * _NS


def _pe_table() -> np.ndarray:
    position = np.arange(_MAX_LEN, dtype=np.float32)[:, None]
    div_term = np.exp(
        np.arange(0, _EMBED, 2, dtype=np.float32) * -(math.log(10000.0) / _EMBED)
    )
    pe = np.zeros((_MAX_LEN, _EMBED), dtype=np.float32)
    pe[:, 0::2] = np.sin(position * div_term)
    pe[:, 1::2] = np.cos(position * div_term)
    return pe


_PE = _pe_table()

_PER_W = 104
_CHUNKS = ((0, 16), (16, 40), (56, 40), (96, 8))


@functools.lru_cache(maxsize=None)
def _build_sc_kernel(n: int, half_l: int):
    mesh = plsc.VectorSubcoreMesh(core_axis_name="c", subcore_axis_name="s")

    @functools.partial(
        pl.kernel,
        out_type=jax.ShapeDtypeStruct((n, _EMBED), jnp.float32),
        mesh=mesh,
        scratch_types=[
            pltpu.VMEM((_PER_W,), jnp.int32),
            pltpu.VMEM((_EMBED,), jnp.float32),
            pltpu.VMEM((_PER_W, _EMBED), jnp.float32),
        ]
        + [pltpu.SemaphoreType.DMA] * (2 * len(_CHUNKS) + 2),
    )
    def sc_kernel(idx_hbm, bias_hbm, table_hbm, out_hbm,
                  idx_v, bias_v, rows_v, *sems):
        nch = len(_CHUNKS)
        gsems, wsems, bsems = sems[:nch], sems[nch:2 * nch], sems[2 * nch:]
        wid = lax.axis_index("s") * _NC + lax.axis_index("c")
        b = wid // 2
        base = b * (2 * half_l) + (wid % 2) * (2 * half_l - _PER_W)
        bias_cp = pltpu.async_copy(bias_hbm.at[b], bias_v, bsems[0])
        idx_cp = pltpu.async_copy(idx_hbm.at[pl.ds(base, _PER_W)], idx_v, bsems[1])
        idx_cp.wait()
        gathers = [
            pltpu.async_copy(
                table_hbm.at[idx_v.at[pl.ds(off, cnt)]],
                rows_v.at[pl.ds(off, cnt)],
                gsems[k],
            )
            for k, (off, cnt) in enumerate(_CHUNKS)
        ]
        bias_cp.wait()
        bias = [bias_v[pl.ds(j * _LANES, _LANES)] for j in range(_EMBED // _LANES)]

        def add_row(i, carry):
            for j in range(_EMBED // _LANES):
                sl = pl.ds(j * _LANES, _LANES)
                rows_v[i, sl] = rows_v[i, sl] + bias[j]
            return carry

        writes = []
        for k, (off, cnt) in enumerate(_CHUNKS):
            gathers[k].wait()
            lax.fori_loop(off, off + cnt, add_row, 0)
            writes.append(
                pltpu.async_copy(
                    rows_v.at[pl.ds(off, cnt)],
                    out_hbm.at[pl.ds(base + off, cnt)],
                    wsems[k],
                )
            )
        for w in writes:
            w.wait()

    return sc_kernel


def kernel(sequence, token_table, segment_weight):
    B, L = sequence.shape
    idx = sequence.astype(jnp.int32).reshape(-1)
    bias2d = jnp.asarray(_PE[:B]) + segment_weight
    out = _build_sc_kernel(B * L, L // 2)(idx, bias2d, token_table)
    return out.reshape(B, L, _EMBED)


# speedup vs baseline: 1.0496x; 1.0015x over previous
import functools
import math

import jax
import jax.numpy as jnp
import numpy as np
from jax import lax
from jax.experimental import pallas as pl
from jax.experimental.pallas import tpu as pltpu
from jax.experimental.pallas import tpu_sc as plsc

_EMBED = 128
_MAX_LEN = 30
_LANES = 16
_NC, _NS = 2, 16
_NW = _NC * _NS


def _pe_table() -> np.ndarray:
    position = np.arange(_MAX_LEN, dtype=np.float32)[:, None]
    div_term = np.exp(
        np.arange(0, _EMBED, 2, dtype=np.float32) * -(math.log(10000.0) / _EMBED)
    )
    pe = np.zeros((_MAX_LEN, _EMBED), dtype=np.float32)
    pe[:, 0::2] = np.sin(position * div_term)
    pe[:, 1::2] = np.cos(position * div_term)
    return pe


_PE = _pe_table()

_PER_W = 104
_CHUNKS = ((0, 48), (48, 48), (96, 8))


@functools.lru_cache(maxsize=None)
def _build_sc_kernel(n: int, half_l: int):
    mesh = plsc.VectorSubcoreMesh(core_axis_name="c", subcore_axis_name="s")

    @functools.partial(
        pl.kernel,
        out_type=jax.ShapeDtypeStruct((n, _EMBED), jnp.float32),
        mesh=mesh,
        scratch_types=[
            pltpu.VMEM((_PER_W,), jnp.int32),
            pltpu.VMEM((_EMBED,), jnp.float32),
            pltpu.VMEM((_PER_W, _EMBED), jnp.float32),
        ]
        + [pltpu.SemaphoreType.DMA] * (2 * len(_CHUNKS) + 2),
    )
    def sc_kernel(idx_hbm, bias_hbm, table_hbm, out_hbm,
                  idx_v, bias_v, rows_v, *sems):
        nch = len(_CHUNKS)
        gsems, wsems, bsems = sems[:nch], sems[nch:2 * nch], sems[2 * nch:]
        wid = lax.axis_index("s") * _NC + lax.axis_index("c")
        b = wid // 2
        base = b * (2 * half_l) + (wid % 2) * (2 * half_l - _PER_W)
        bias_cp = pltpu.async_copy(bias_hbm.at[b], bias_v, bsems[0])
        idx_cp = pltpu.async_copy(idx_hbm.at[pl.ds(base, _PER_W)], idx_v, bsems[1])
        idx_cp.wait()
        gathers = [
            pltpu.async_copy(
                table_hbm.at[idx_v.at[pl.ds(off, cnt)]],
                rows_v.at[pl.ds(off, cnt)],
                gsems[k],
            )
            for k, (off, cnt) in enumerate(_CHUNKS)
        ]
        bias_cp.wait()
        bias = [bias_v[pl.ds(j * _LANES, _LANES)] for j in range(_EMBED // _LANES)]

        def add_row(i, carry):
            for j in range(_EMBED // _LANES):
                sl = pl.ds(j * _LANES, _LANES)
                rows_v[i, sl] = rows_v[i, sl] + bias[j]
            return carry

        writes = []
        for k, (off, cnt) in enumerate(_CHUNKS):
            gathers[k].wait()
            lax.fori_loop(off, off + cnt, add_row, 0)
            writes.append(
                pltpu.async_copy(
                    rows_v.at[pl.ds(off, cnt)],
                    out_hbm.at[pl.ds(base + off, cnt)],
                    wsems[k],
                )
            )
        for w in writes:
            w.wait()

    return sc_kernel


def kernel(sequence, token_table, segment_weight):
    B, L = sequence.shape
    idx = sequence.astype(jnp.int32).reshape(-1)
    bias2d = jnp.asarray(_PE[:B]) + segment_weight
    out = _build_sc_kernel(B * L, L // 2)(idx, bias2d, token_table)
    return out.reshape(B, L, _EMBED)
